# initial kernel scaffold (unmeasured)
import jax
import jax.numpy as jnp
from jax import lax
from jax.experimental import pallas as pl
from jax.experimental.pallas import tpu as pltpu

N_DEV = 4
M, N = 8192, 4096
CHUNK = M // N_DEV
TILE = 512
N_TILES = CHUNK // TILE


def _ar_relu(z):

    def body(z_ref, o_ref, send_buf, recv_bufs, vmem_a, vmem_b,
             send_sems, rs_sems, ag_sems, local_sems):
        my = lax.axis_index("i")
        left = lax.rem(my - 1 + N_DEV, N_DEV)
        right = lax.rem(my + 1, N_DEV)

        barrier = pltpu.get_barrier_semaphore()
        for nbr in (left, right):
            pl.semaphore_signal(barrier, inc=1, device_id=(nbr,),
                                device_id_type=pl.DeviceIdType.MESH)
        pl.semaphore_wait(barrier, 2)

        def rows(c, t=0, size=CHUNK):
            return pl.ds(c * CHUNK + t * TILE, size)

        for s in range(N_DEV - 1):
            c_send = lax.rem(my - s + N_DEV, N_DEV)
            c_recv = lax.rem(my - s - 1 + N_DEV, N_DEV)
            if s == 0:
                src = z_ref.at[rows(c_send), :]
            else:
                src = send_buf
            rdma = pltpu.make_async_remote_copy(
                src_ref=src,
                dst_ref=recv_bufs.at[s],
                send_sem=send_sems.at[s],
                recv_sem=rs_sems.at[s],
                device_id=(right,),
                device_id_type=pl.DeviceIdType.MESH,
            )
            rdma.start()
            rdma.wait()

            for t in range(N_TILES):
                cp_a = pltpu.make_async_copy(
                    recv_bufs.at[s, pl.ds(t * TILE, TILE), :],
                    vmem_a, local_sems.at[0])
                cp_b = pltpu.make_async_copy(
                    z_ref.at[rows(c_recv, t, TILE), :],
                    vmem_b, local_sems.at[1])
                cp_a.start()
                cp_b.start()
                cp_a.wait()
                cp_b.wait()
                if s < N_DEV - 2:
                    vmem_a[...] = vmem_a[...] + vmem_b[...]
                    cp_out = pltpu.make_async_copy(
                        vmem_a, send_buf.at[pl.ds(t * TILE, TILE), :],
                        local_sems.at[0])
                else:
                    vmem_a[...] = jnp.maximum(vmem_a[...] + vmem_b[...], 0.0)
                    cp_out = pltpu.make_async_copy(
                        vmem_a, o_ref.at[rows(c_recv, t, TILE), :],
                        local_sems.at[0])
                cp_out.start()
                cp_out.wait()

        for t in range(N_DEV - 1):
            c_fwd = lax.rem(my + 1 - t + 2 * N_DEV, N_DEV)
            rdma = pltpu.make_async_remote_copy(
                src_ref=o_ref.at[rows(c_fwd), :],
                dst_ref=o_ref.at[rows(c_fwd), :],
                send_sem=send_sems.at[N_DEV - 1 + t],
                recv_sem=ag_sems.at[t],
                device_id=(right,),
                device_id_type=pl.DeviceIdType.MESH,
            )
            rdma.start()
            rdma.wait()

    return pl.pallas_call(
        body,
        out_shape=jax.ShapeDtypeStruct((M, N), jnp.float32),
        in_specs=[pl.BlockSpec(memory_space=pl.ANY)],
        out_specs=pl.BlockSpec(memory_space=pl.ANY),
        scratch_shapes=[
            pltpu.HBM((CHUNK, N), jnp.float32),
            pltpu.HBM((N_DEV - 1, CHUNK, N), jnp.float32),
            pltpu.VMEM((TILE, N), jnp.float32),
            pltpu.VMEM((TILE, N), jnp.float32),
            pltpu.SemaphoreType.DMA((2 * (N_DEV - 1),)),
            pltpu.SemaphoreType.DMA((N_DEV - 1,)),
            pltpu.SemaphoreType.DMA((N_DEV - 1,)),
            pltpu.SemaphoreType.DMA((2,)),
        ],
        compiler_params=pltpu.CompilerParams(collective_id=0),
    )(z)


def kernel(x, w_mat):
    z = jnp.dot(x, w_mat, preferred_element_type=jnp.float32)
    return _ar_relu(z)


# baseline (device time: 2569930 ns/iter reference)
import jax
import jax.numpy as jnp
from jax import lax
from jax.experimental import pallas as pl
from jax.experimental.pallas import tpu as pltpu

N_DEV = 4
M, N = 8192, 4096
CHUNK = M // N_DEV
TILE = 512
N_TILES = CHUNK // TILE


def _ar_relu(z):

    def body(z_ref, o_ref, send_buf, recv_bufs, vmem_a, vmem_b,
             send_sems, rs_sems, ag_sems, local_sems):
        my = lax.axis_index("i")
        left = lax.rem(my - 1 + N_DEV, N_DEV)
        right = lax.rem(my + 1, N_DEV)

        barrier = pltpu.get_barrier_semaphore()
        for nbr in (left, right):
            pl.semaphore_signal(barrier, inc=1, device_id=(nbr,),
                                device_id_type=pl.DeviceIdType.MESH)
        pl.semaphore_wait(barrier, 2)

        def rows(c, t=0, size=CHUNK):
            return pl.ds(c * CHUNK + t * TILE, size)

        for s in range(N_DEV - 1):
            c_send = lax.rem(my - s + N_DEV, N_DEV)
            c_recv = lax.rem(my - s - 1 + N_DEV, N_DEV)
            if s == 0:
                src = z_ref.at[rows(c_send), :]
            else:
                src = send_buf
            rdma = pltpu.make_async_remote_copy(
                src_ref=src,
                dst_ref=recv_bufs.at[s],
                send_sem=send_sems.at[s],
                recv_sem=rs_sems.at[s],
                device_id=(right,),
                device_id_type=pl.DeviceIdType.MESH,
            )
            rdma.start()
            rdma.wait()

            for t in range(N_TILES):
                cp_a = pltpu.make_async_copy(
                    recv_bufs.at[s, pl.ds(t * TILE, TILE), :],
                    vmem_a, local_sems.at[0])
                cp_b = pltpu.make_async_copy(
                    z_ref.at[rows(c_recv, t, TILE), :],
                    vmem_b, local_sems.at[1])
                cp_a.start()
                cp_b.start()
                cp_a.wait()
                cp_b.wait()
                if s < N_DEV - 2:
                    vmem_a[...] = vmem_a[...] + vmem_b[...]
                    cp_out = pltpu.make_async_copy(
                        vmem_a, send_buf.at[pl.ds(t * TILE, TILE), :],
                        local_sems.at[0])
                else:
                    vmem_a[...] = jnp.maximum(vmem_a[...] + vmem_b[...], 0.0)
                    cp_out = pltpu.make_async_copy(
                        vmem_a, o_ref.at[rows(c_recv, t, TILE), :],
                        local_sems.at[0])
                cp_out.start()
                cp_out.wait()

        for t in range(N_DEV - 1):
            c_fwd = lax.rem(my + 1 - t + 2 * N_DEV, N_DEV)
            rdma = pltpu.make_async_remote_copy(
                src_ref=o_ref.at[rows(c_fwd), :],
                dst_ref=o_ref.at[rows(c_fwd), :],
                send_sem=send_sems.at[N_DEV - 1 + t],
                recv_sem=ag_sems.at[t],
                device_id=(right,),
                device_id_type=pl.DeviceIdType.MESH,
            )
            rdma.start()
            rdma.wait()

    out, _, _ = pl.pallas_call(
        body,
        out_shape=(
            jax.ShapeDtypeStruct((M, N), jnp.float32),
            jax.ShapeDtypeStruct((CHUNK, N), jnp.float32),
            jax.ShapeDtypeStruct((N_DEV - 1, CHUNK, N), jnp.float32),
        ),
        in_specs=[pl.BlockSpec(memory_space=pl.ANY)],
        out_specs=(
            pl.BlockSpec(memory_space=pl.ANY),
            pl.BlockSpec(memory_space=pl.ANY),
            pl.BlockSpec(memory_space=pl.ANY),
        ),
        scratch_shapes=[
            pltpu.VMEM((TILE, N), jnp.float32),
            pltpu.VMEM((TILE, N), jnp.float32),
            pltpu.SemaphoreType.DMA((2 * (N_DEV - 1),)),
            pltpu.SemaphoreType.DMA((N_DEV - 1,)),
            pltpu.SemaphoreType.DMA((N_DEV - 1,)),
            pltpu.SemaphoreType.DMA((2,)),
        ],
        compiler_params=pltpu.CompilerParams(collective_id=0),
    )(z)
    return out


def kernel(x, w_mat):
    z = jnp.dot(x, w_mat, preferred_element_type=jnp.float32)
    return _ar_relu(z)


# device time: 1513303 ns/iter; 1.6982x vs baseline; 1.6982x over previous
import jax
import jax.numpy as jnp
from jax import lax
from jax.experimental import pallas as pl
from jax.experimental.pallas import tpu as pltpu

N_DEV = 4
M, N = 8192, 4096
CHUNK = M // N_DEV
HALF = N // 2
TILE = 512
N_TILES = CHUNK // TILE
N_HOPS = N_DEV - 1


def _ar_relu(z):

    def body(z_ref, o_ref, send_bufs, recv_bufs, vmem_a, vmem_b,
             send_sems, rs_sems, ag_sems, local_sems):
        my = lax.axis_index("i")
        left = lax.rem(my - 1 + N_DEV, N_DEV)
        right = lax.rem(my + 1, N_DEV)
        tgt = (right, left)

        barrier = pltpu.get_barrier_semaphore()
        for nbr in (left, right):
            pl.semaphore_signal(barrier, inc=1, device_id=(nbr,),
                                device_id_type=pl.DeviceIdType.MESH)
        pl.semaphore_wait(barrier, 2)

        def rows(c, t=0, size=CHUNK):
            return pl.ds(c * CHUNK + t * TILE, size)

        def cols(d):
            return pl.ds(d * HALF, HALF)

        for s in range(N_HOPS):
            rdmas = []
            for d in (0, 1):
                c_send = lax.rem(my + (s if d else -s) + 2 * N_DEV, N_DEV)
                if s == 0:
                    src = z_ref.at[rows(c_send), cols(d)]
                else:
                    src = send_bufs.at[d]
                rdmas.append(pltpu.make_async_remote_copy(
                    src_ref=src,
                    dst_ref=recv_bufs.at[d, s],
                    send_sem=send_sems.at[d, s],
                    recv_sem=rs_sems.at[d, s],
                    device_id=(tgt[d],),
                    device_id_type=pl.DeviceIdType.MESH,
                ))
            for r in rdmas:
                r.start()
            for d in (0, 1):
                rdmas[d].wait()
                c_recv = lax.rem(my + (s + 1 if d else -s - 1) + 2 * N_DEV,
                                 N_DEV)
                for t in range(N_TILES):
                    cp_a = pltpu.make_async_copy(
                        recv_bufs.at[d, s, pl.ds(t * TILE, TILE), :],
                        vmem_a, local_sems.at[0])
                    cp_b = pltpu.make_async_copy(
                        z_ref.at[rows(c_recv, t, TILE), cols(d)],
                        vmem_b, local_sems.at[1])
                    cp_a.start()
                    cp_b.start()
                    cp_a.wait()
                    cp_b.wait()
                    if s < N_HOPS - 1:
                        vmem_a[...] = vmem_a[...] + vmem_b[...]
                        cp_out = pltpu.make_async_copy(
                            vmem_a, send_bufs.at[d, pl.ds(t * TILE, TILE), :],
                            local_sems.at[0])
                    else:
                        vmem_a[...] = jnp.maximum(
                            vmem_a[...] + vmem_b[...], 0.0)
                        cp_out = pltpu.make_async_copy(
                            vmem_a, o_ref.at[rows(c_recv, t, TILE), cols(d)],
                            local_sems.at[0])
                    cp_out.start()
                    cp_out.wait()

        for t in range(N_HOPS):
            rdmas = []
            for d in (0, 1):
                c_fwd = lax.rem(my + (t - 1 if d else 1 - t) + 2 * N_DEV,
                                N_DEV)
                rdmas.append(pltpu.make_async_remote_copy(
                    src_ref=o_ref.at[rows(c_fwd), cols(d)],
                    dst_ref=o_ref.at[rows(c_fwd), cols(d)],
                    send_sem=send_sems.at[d, N_HOPS + t],
                    recv_sem=ag_sems.at[d, t],
                    device_id=(tgt[d],),
                    device_id_type=pl.DeviceIdType.MESH,
                ))
            for r in rdmas:
                r.start()
            for r in rdmas:
                r.wait()

    out, _, _ = pl.pallas_call(
        body,
        out_shape=(
            jax.ShapeDtypeStruct((M, N), jnp.float32),
            jax.ShapeDtypeStruct((2, CHUNK, HALF), jnp.float32),
            jax.ShapeDtypeStruct((2, N_HOPS, CHUNK, HALF), jnp.float32),
        ),
        in_specs=[pl.BlockSpec(memory_space=pl.ANY)],
        out_specs=(
            pl.BlockSpec(memory_space=pl.ANY),
            pl.BlockSpec(memory_space=pl.ANY),
            pl.BlockSpec(memory_space=pl.ANY),
        ),
        scratch_shapes=[
            pltpu.VMEM((TILE, HALF), jnp.float32),
            pltpu.VMEM((TILE, HALF), jnp.float32),
            pltpu.SemaphoreType.DMA((2, 2 * N_HOPS)),
            pltpu.SemaphoreType.DMA((2, N_HOPS)),
            pltpu.SemaphoreType.DMA((2, N_HOPS)),
            pltpu.SemaphoreType.DMA((2,)),
        ],
        compiler_params=pltpu.CompilerParams(collective_id=0),
    )(z)
    return out


def kernel(x, w_mat):
    z = jnp.dot(x, w_mat, preferred_element_type=jnp.float32)
    return _ar_relu(z)


# device time: 1362437 ns/iter; 1.8863x vs baseline; 1.1107x over previous
import jax
import jax.numpy as jnp
from jax import lax
from jax.experimental import pallas as pl
from jax.experimental.pallas import tpu as pltpu

N_DEV = 4
M, N = 8192, 4096
CHUNK = M // N_DEV
HALF = N // 2
TILE = 512
N_TILES = CHUNK // TILE
N_HOPS = N_DEV - 1


def _ar_relu(z):

    def body(z_ref, o_ref, send_bufs, recv_bufs, vmem_a, vmem_b,
             rs_send_sems, rs_recv_sems, ag_send_sems, ag_recv_sems,
             local_sems):
        my = lax.axis_index("i")
        left = lax.rem(my - 1 + N_DEV, N_DEV)
        right = lax.rem(my + 1, N_DEV)
        tgt = (right, left)

        def c_rs_send(d, s):
            return lax.rem(my + (s if d else -s) + 2 * N_DEV, N_DEV)

        def c_rs_recv(d, s):
            return lax.rem(my + (s + 1 if d else -s - 1) + 2 * N_DEV, N_DEV)

        def c_ag(d, h):
            return lax.rem(my + (h - 1 if d else 1 - h) + 2 * N_DEV, N_DEV)

        def rows(c, t, size=TILE):
            return pl.ds(c * CHUNK + t * TILE, size)

        def cols(d):
            return pl.ds(d * HALF, HALF)

        rs = [[[None] * N_TILES for _ in range(N_HOPS)] for _ in range(2)]
        ag = [[[None] * N_TILES for _ in range(N_HOPS)] for _ in range(2)]
        for d in (0, 1):
            for s in range(N_HOPS):
                for t in range(N_TILES):
                    if s == 0:
                        src = z_ref.at[rows(c_rs_send(d, 0), t), cols(d)]
                    else:
                        src = send_bufs.at[d, pl.ds(t * TILE, TILE), :]
                    rs[d][s][t] = pltpu.make_async_remote_copy(
                        src_ref=src,
                        dst_ref=recv_bufs.at[d, s, pl.ds(t * TILE, TILE), :],
                        send_sem=rs_send_sems.at[d, s, t],
                        recv_sem=rs_recv_sems.at[d, s, t],
                        device_id=(tgt[d],),
                        device_id_type=pl.DeviceIdType.MESH,
                    )
            for h in range(N_HOPS):
                for t in range(N_TILES):
                    sl = (rows(c_ag(d, h), t), cols(d))
                    ag[d][h][t] = pltpu.make_async_remote_copy(
                        src_ref=o_ref.at[sl],
                        dst_ref=o_ref.at[sl],
                        send_sem=ag_send_sems.at[d, h, t],
                        recv_sem=ag_recv_sems.at[d, h, t],
                        device_id=(tgt[d],),
                        device_id_type=pl.DeviceIdType.MESH,
                    )

        barrier = pltpu.get_barrier_semaphore()
        for nbr in (left, right):
            pl.semaphore_signal(barrier, inc=1, device_id=(nbr,),
                                device_id_type=pl.DeviceIdType.MESH)
        pl.semaphore_wait(barrier, 2)

        for d in (0, 1):
            for t in range(N_TILES):
                rs[d][0][t].start()
        for s in range(N_HOPS):
            for t in range(N_TILES):
                for d in (0, 1):
                    rs[d][s][t].wait()
                    c = c_rs_recv(d, s)
                    cp_a = pltpu.make_async_copy(
                        recv_bufs.at[d, s, pl.ds(t * TILE, TILE), :],
                        vmem_a.at[d], local_sems.at[d, 0])
                    cp_b = pltpu.make_async_copy(
                        z_ref.at[rows(c, t), cols(d)],
                        vmem_b.at[d], local_sems.at[d, 1])
                    cp_a.start()
                    cp_b.start()
                    cp_a.wait()
                    cp_b.wait()
                    if s < N_HOPS - 1:
                        vmem_a[d] = vmem_a[d] + vmem_b[d]
                        cp_out = pltpu.make_async_copy(
                            vmem_a.at[d],
                            send_bufs.at[d, pl.ds(t * TILE, TILE), :],
                            local_sems.at[d, 0])
                        cp_out.start()
                        cp_out.wait()
                        rs[d][s + 1][t].start()
                    else:
                        vmem_a[d] = jnp.maximum(vmem_a[d] + vmem_b[d], 0.0)
                        cp_out = pltpu.make_async_copy(
                            vmem_a.at[d], o_ref.at[rows(c, t), cols(d)],
                            local_sems.at[d, 0])
                        cp_out.start()
                        cp_out.wait()
                        ag[d][0][t].start()

        for h in range(N_HOPS):
            for t in range(N_TILES):
                for d in (0, 1):
                    ag[d][h][t].wait()
                    if h < N_HOPS - 1:
                        ag[d][h + 1][t].start()

    out, _, _ = pl.pallas_call(
        body,
        out_shape=(
            jax.ShapeDtypeStruct((M, N), jnp.float32),
            jax.ShapeDtypeStruct((2, CHUNK, HALF), jnp.float32),
            jax.ShapeDtypeStruct((2, N_HOPS, CHUNK, HALF), jnp.float32),
        ),
        in_specs=[pl.BlockSpec(memory_space=pl.ANY)],
        out_specs=(
            pl.BlockSpec(memory_space=pl.ANY),
            pl.BlockSpec(memory_space=pl.ANY),
            pl.BlockSpec(memory_space=pl.ANY),
        ),
        scratch_shapes=[
            pltpu.VMEM((2, TILE, HALF), jnp.float32),
            pltpu.VMEM((2, TILE, HALF), jnp.float32),
            pltpu.SemaphoreType.DMA((2, N_HOPS, N_TILES)),
            pltpu.SemaphoreType.DMA((2, N_HOPS, N_TILES)),
            pltpu.SemaphoreType.DMA((2, N_HOPS, N_TILES)),
            pltpu.SemaphoreType.DMA((2, N_HOPS, N_TILES)),
            pltpu.SemaphoreType.DMA((2, 2)),
        ],
        compiler_params=pltpu.CompilerParams(collective_id=0),
    )(z)
    return out


def kernel(x, w_mat):
    z = jnp.dot(x, w_mat, preferred_element_type=jnp.float32)
    return _ar_relu(z)


# device time: 1243114 ns/iter; 2.0673x vs baseline; 1.0960x over previous
import jax
import jax.numpy as jnp
from jax import lax
from jax.experimental import pallas as pl
from jax.experimental.pallas import tpu as pltpu

N_DEV = 4
M, N = 8192, 4096
K = 2048
CHUNK = M // N_DEV
HALF = N // 2
TILE = 512
N_TILES = CHUNK // TILE
N_HOPS = N_DEV - 1
TILE_G = 1024
GPT = TILE_G // TILE
ATILE = 256


def kernel(x, w_mat):
    def body(x_ref, w_ref, o_ref, z_buf, send_bufs, recv_bufs,
             w_vmem, x_vmem, zt_vmem, vmem_a, vmem_b,
             rs_send_sems, rs_recv_sems, ag_send_sems, ag_recv_sems,
             local_sems, gemm_sems):
        my = lax.axis_index("i")
        left = lax.rem(my - 1 + N_DEV, N_DEV)
        right = lax.rem(my + 1, N_DEV)
        tgt = (right, left)

        def c_rs_send(d, s):
            return lax.rem(my + (s if d else -s) + 2 * N_DEV, N_DEV)

        def c_rs_recv(d, s):
            return lax.rem(my + (s + 1 if d else -s - 1) + 2 * N_DEV, N_DEV)

        def c_ag(d, h):
            return lax.rem(my + (h - 1 if d else 1 - h) + 2 * N_DEV, N_DEV)

        def rows(c, t, size=TILE):
            return pl.ds(c * CHUNK + t * TILE, size)

        def cols(d):
            return pl.ds(d * HALF, HALF)

        rs = [[[None] * N_TILES for _ in range(N_HOPS)] for _ in range(2)]
        ag = [[[None] * N_TILES for _ in range(N_HOPS)] for _ in range(2)]
        for d in (0, 1):
            for s in range(N_HOPS):
                for t in range(N_TILES):
                    if s == 0:
                        src = z_buf.at[rows(c_rs_send(d, 0), t), cols(d)]
                    else:
                        src = send_bufs.at[d, pl.ds(t * TILE, TILE), :]
                    rs[d][s][t] = pltpu.make_async_remote_copy(
                        src_ref=src,
                        dst_ref=recv_bufs.at[d, s, pl.ds(t * TILE, TILE), :],
                        send_sem=rs_send_sems.at[d, s, t],
                        recv_sem=rs_recv_sems.at[d, s, t],
                        device_id=(tgt[d],),
                        device_id_type=pl.DeviceIdType.MESH,
                    )
            for h in range(N_HOPS):
                for t in range(N_TILES):
                    sl = (rows(c_ag(d, h), t), cols(d))
                    ag[d][h][t] = pltpu.make_async_remote_copy(
                        src_ref=o_ref.at[sl],
                        dst_ref=o_ref.at[sl],
                        send_sem=ag_send_sems.at[d, h, t],
                        recv_sem=ag_recv_sems.at[d, h, t],
                        device_id=(tgt[d],),
                        device_id_type=pl.DeviceIdType.MESH,
                    )

        cp_w = pltpu.make_async_copy(w_ref, w_vmem, gemm_sems.at[0])
        cp_w.start()

        barrier = pltpu.get_barrier_semaphore()
        for nbr in (left, right):
            pl.semaphore_signal(barrier, inc=1, device_id=(nbr,),
                                device_id_type=pl.DeviceIdType.MESH)
        pl.semaphore_wait(barrier, 2)
        cp_w.wait()

        def gemm_body(j, carry):
            delta = jnp.where(
                j < 2, 0,
                jnp.where(j >= 6, 2,
                          jnp.where((j == 2) | (j == 4), -1, 1)))
            g = jnp.where(j < 2, j, jnp.where(j >= 6, j - 6, (j - 2) // 2))
            c = lax.rem(my + delta + 2 * N_DEV, N_DEV)
            off = c * CHUNK + g * TILE_G
            cp_x = pltpu.make_async_copy(
                x_ref.at[pl.ds(off, TILE_G), :], x_vmem, gemm_sems.at[0])
            cp_x.start()
            cp_x.wait()
            zt_vmem[...] = jnp.dot(x_vmem[...], w_vmem[...],
                                   preferred_element_type=jnp.float32)
            cp_z = pltpu.make_async_copy(
                zt_vmem, z_buf.at[pl.ds(off, TILE_G), :], gemm_sems.at[1])
            cp_z.start()
            cp_z.wait()

            @pl.when(j == 0)
            def _():
                for t in range(GPT):
                    rs[0][0][t].start()
                    rs[1][0][t].start()

            @pl.when(j == 1)
            def _():
                for t in range(GPT, 2 * GPT):
                    rs[0][0][t].start()
                    rs[1][0][t].start()

            return carry

        lax.fori_loop(0, 2 * N_DEV, gemm_body, 0)

        for s in range(N_HOPS):
            for t in range(N_TILES):
                for d in (0, 1):
                    rs[d][s][t].wait()
                    c = c_rs_recv(d, s)
                    for p in range(TILE // ATILE):
                        off = pl.ds(t * TILE + p * ATILE, ATILE)
                        zoff = pl.ds(c * CHUNK + t * TILE + p * ATILE, ATILE)
                        cp_a = pltpu.make_async_copy(
                            recv_bufs.at[d, s, off, :],
                            vmem_a, local_sems.at[0])
                        cp_b = pltpu.make_async_copy(
                            z_buf.at[zoff, cols(d)],
                            vmem_b, local_sems.at[1])
                        cp_a.start()
                        cp_b.start()
                        cp_a.wait()
                        cp_b.wait()
                        if s < N_HOPS - 1:
                            vmem_a[...] = vmem_a[...] + vmem_b[...]
                            cp_out = pltpu.make_async_copy(
                                vmem_a, send_bufs.at[d, off, :],
                                local_sems.at[0])
                        else:
                            vmem_a[...] = jnp.maximum(
                                vmem_a[...] + vmem_b[...], 0.0)
                            cp_out = pltpu.make_async_copy(
                                vmem_a, o_ref.at[zoff, cols(d)],
                                local_sems.at[0])
                        cp_out.start()
                        cp_out.wait()
                    if s < N_HOPS - 1:
                        rs[d][s + 1][t].start()
                    else:
                        ag[d][0][t].start()

        for h in range(N_HOPS):
            for t in range(N_TILES):
                for d in (0, 1):
                    ag[d][h][t].wait()
                    if h < N_HOPS - 1:
                        ag[d][h + 1][t].start()

    out, _, _, _ = pl.pallas_call(
        body,
        out_shape=(
            jax.ShapeDtypeStruct((M, N), jnp.float32),
            jax.ShapeDtypeStruct((M, N), jnp.float32),
            jax.ShapeDtypeStruct((2, CHUNK, HALF), jnp.float32),
            jax.ShapeDtypeStruct((2, N_HOPS, CHUNK, HALF), jnp.float32),
        ),
        in_specs=[
            pl.BlockSpec(memory_space=pl.ANY),
            pl.BlockSpec(memory_space=pl.ANY),
        ],
        out_specs=(
            pl.BlockSpec(memory_space=pl.ANY),
            pl.BlockSpec(memory_space=pl.ANY),
            pl.BlockSpec(memory_space=pl.ANY),
            pl.BlockSpec(memory_space=pl.ANY),
        ),
        scratch_shapes=[
            pltpu.VMEM((K, N), jnp.float32),
            pltpu.VMEM((TILE_G, K), jnp.float32),
            pltpu.VMEM((TILE_G, N), jnp.float32),
            pltpu.VMEM((ATILE, HALF), jnp.float32),
            pltpu.VMEM((ATILE, HALF), jnp.float32),
            pltpu.SemaphoreType.DMA((2, N_HOPS, N_TILES)),
            pltpu.SemaphoreType.DMA((2, N_HOPS, N_TILES)),
            pltpu.SemaphoreType.DMA((2, N_HOPS, N_TILES)),
            pltpu.SemaphoreType.DMA((2, N_HOPS, N_TILES)),
            pltpu.SemaphoreType.DMA((2,)),
            pltpu.SemaphoreType.DMA((2,)),
        ],
        compiler_params=pltpu.CompilerParams(
            collective_id=0, vmem_limit_bytes=100 * 1024 * 1024),
    )(x, w_mat)
    return out


# device time: 1207938 ns/iter; 2.1275x vs baseline; 1.0291x over previous
import jax
import jax.numpy as jnp
from jax import lax
from jax.experimental import pallas as pl
from jax.experimental.pallas import tpu as pltpu

N_DEV = 4
M, N = 8192, 4096
K = 2048
CHUNK = M // N_DEV
HALF = N // 2
TILE = 512
N_TILES = CHUNK // TILE
N_HOPS = N_DEV - 1
TILE_G = 1024
GPT = TILE_G // TILE
ATILE = 256


def kernel(x, w_mat):
    def body(x_ref, w_ref, o_ref, z_buf, send_bufs, recv_bufs,
             w_vmem, x_vmem, zt_vmem, vmem_a, vmem_b,
             rs_send_sems, rs_recv_sems, ag_send_sems, ag_recv_sems,
             local_sems, gemm_sems):
        my = lax.axis_index("i")
        left = lax.rem(my - 1 + N_DEV, N_DEV)
        right = lax.rem(my + 1, N_DEV)
        tgt = (right, left)

        def c_rs_send(d, s):
            return lax.rem(my + (s if d else -s) + 2 * N_DEV, N_DEV)

        def c_rs_recv(d, s):
            return lax.rem(my + (s + 1 if d else -s - 1) + 2 * N_DEV, N_DEV)

        def c_ag(d, h):
            return lax.rem(my + (h - 1 if d else 1 - h) + 2 * N_DEV, N_DEV)

        def rows(c, t, size=TILE):
            return pl.ds(c * CHUNK + t * TILE, size)

        def cols(d):
            return pl.ds(d * HALF, HALF)

        rs = [[[None] * N_TILES for _ in range(N_HOPS)] for _ in range(2)]
        ag = [[[None] * N_TILES for _ in range(N_HOPS)] for _ in range(2)]
        for d in (0, 1):
            for s in range(N_HOPS):
                for t in range(N_TILES):
                    if s == 0:
                        src = z_buf.at[rows(c_rs_send(d, 0), t), cols(d)]
                    else:
                        src = send_bufs.at[d, pl.ds(t * TILE, TILE), :]
                    rs[d][s][t] = pltpu.make_async_remote_copy(
                        src_ref=src,
                        dst_ref=recv_bufs.at[d, s, pl.ds(t * TILE, TILE), :],
                        send_sem=rs_send_sems.at[d, s, t],
                        recv_sem=rs_recv_sems.at[d, s, t],
                        device_id=(tgt[d],),
                        device_id_type=pl.DeviceIdType.MESH,
                    )
            for h in range(N_HOPS):
                for t in range(N_TILES):
                    sl = (rows(c_ag(d, h), t), cols(d))
                    ag[d][h][t] = pltpu.make_async_remote_copy(
                        src_ref=o_ref.at[sl],
                        dst_ref=o_ref.at[sl],
                        send_sem=ag_send_sems.at[d, h, t],
                        recv_sem=ag_recv_sems.at[d, h, t],
                        device_id=(tgt[d],),
                        device_id_type=pl.DeviceIdType.MESH,
                    )

        cp_w = pltpu.make_async_copy(w_ref, w_vmem, gemm_sems.at[0])
        cp_w.start()

        barrier = pltpu.get_barrier_semaphore()
        for nbr in (left, right):
            pl.semaphore_signal(barrier, inc=1, device_id=(nbr,),
                                device_id_type=pl.DeviceIdType.MESH)
        pl.semaphore_wait(barrier, 2)
        cp_w.wait()

        def _gemm_tile_at(delta, g, j, send_hop0):
            c = lax.rem(my + delta + 2 * N_DEV, N_DEV)
            off = c * CHUNK + g * TILE_G
            cp_x = pltpu.make_async_copy(
                x_ref.at[pl.ds(off, TILE_G), :], x_vmem, gemm_sems.at[0])
            cp_x.start()
            cp_x.wait()
            zt_vmem[...] = jnp.dot(x_vmem[...], w_vmem[...],
                                   preferred_element_type=jnp.float32)
            cp_z = pltpu.make_async_copy(
                zt_vmem, z_buf.at[pl.ds(off, TILE_G), :], gemm_sems.at[1])
            cp_z.start()
            cp_z.wait()
            if send_hop0:
                @pl.when(j == 0)
                def _():
                    for t in range(GPT):
                        rs[0][0][t].start()
                        rs[1][0][t].start()

                @pl.when(j == 1)
                def _():
                    for t in range(GPT, 2 * GPT):
                        rs[0][0][t].start()
                        rs[1][0][t].start()

        def gemm_body1(j, carry):
            delta = jnp.where(j < 2, 0, jnp.where(j == 2, -1, 1))
            g = jnp.where(j < 2, j, 0)
            _gemm_tile_at(delta, g, j, send_hop0=True)
            return carry

        def gemm_body2(j, carry):
            delta = jnp.where(j == 0, -1, jnp.where(j == 1, 1, 2))
            g = jnp.where(j < 2, 1, j - 2)
            _gemm_tile_at(delta, g, j, send_hop0=False)
            return carry

        lax.fori_loop(0, 4, gemm_body1, 0)

        def emit_add(s, t):
                for d in (0, 1):
                    rs[d][s][t].wait()
                    c = c_rs_recv(d, s)
                    for p in range(TILE // ATILE):
                        off = pl.ds(t * TILE + p * ATILE, ATILE)
                        zoff = pl.ds(c * CHUNK + t * TILE + p * ATILE, ATILE)
                        cp_a = pltpu.make_async_copy(
                            recv_bufs.at[d, s, off, :],
                            vmem_a, local_sems.at[0])
                        cp_b = pltpu.make_async_copy(
                            z_buf.at[zoff, cols(d)],
                            vmem_b, local_sems.at[1])
                        cp_a.start()
                        cp_b.start()
                        cp_a.wait()
                        cp_b.wait()
                        if s < N_HOPS - 1:
                            vmem_a[...] = vmem_a[...] + vmem_b[...]
                            cp_out = pltpu.make_async_copy(
                                vmem_a, send_bufs.at[d, off, :],
                                local_sems.at[0])
                        else:
                            vmem_a[...] = jnp.maximum(
                                vmem_a[...] + vmem_b[...], 0.0)
                            cp_out = pltpu.make_async_copy(
                                vmem_a, o_ref.at[zoff, cols(d)],
                                local_sems.at[0])
                        cp_out.start()
                        cp_out.wait()
                    if s < N_HOPS - 1:
                        rs[d][s + 1][t].start()
                    else:
                        ag[d][0][t].start()

        emit_add(0, 0)
        emit_add(0, 1)
        lax.fori_loop(0, 4, gemm_body2, 0)
        emit_add(0, 2)
        emit_add(0, 3)
        for s in range(1, N_HOPS):
            for t in range(N_TILES):
                emit_add(s, t)

        for h in range(N_HOPS):
            for t in range(N_TILES):
                for d in (0, 1):
                    ag[d][h][t].wait()
                    if h < N_HOPS - 1:
                        ag[d][h + 1][t].start()

    out, _, _, _ = pl.pallas_call(
        body,
        out_shape=(
            jax.ShapeDtypeStruct((M, N), jnp.float32),
            jax.ShapeDtypeStruct((M, N), jnp.float32),
            jax.ShapeDtypeStruct((2, CHUNK, HALF), jnp.float32),
            jax.ShapeDtypeStruct((2, N_HOPS, CHUNK, HALF), jnp.float32),
        ),
        in_specs=[
            pl.BlockSpec(memory_space=pl.ANY),
            pl.BlockSpec(memory_space=pl.ANY),
        ],
        out_specs=(
            pl.BlockSpec(memory_space=pl.ANY),
            pl.BlockSpec(memory_space=pl.ANY),
            pl.BlockSpec(memory_space=pl.ANY),
            pl.BlockSpec(memory_space=pl.ANY),
        ),
        scratch_shapes=[
            pltpu.VMEM((K, N), jnp.float32),
            pltpu.VMEM((TILE_G, K), jnp.float32),
            pltpu.VMEM((TILE_G, N), jnp.float32),
            pltpu.VMEM((ATILE, HALF), jnp.float32),
            pltpu.VMEM((ATILE, HALF), jnp.float32),
            pltpu.SemaphoreType.DMA((2, N_HOPS, N_TILES)),
            pltpu.SemaphoreType.DMA((2, N_HOPS, N_TILES)),
            pltpu.SemaphoreType.DMA((2, N_HOPS, N_TILES)),
            pltpu.SemaphoreType.DMA((2, N_HOPS, N_TILES)),
            pltpu.SemaphoreType.DMA((2,)),
            pltpu.SemaphoreType.DMA((2,)),
        ],
        compiler_params=pltpu.CompilerParams(
            collective_id=0, vmem_limit_bytes=100 * 1024 * 1024),
    )(x, w_mat)
    return out


# device time: 1205304 ns/iter; 2.1322x vs baseline; 1.0022x over previous
import jax
import jax.numpy as jnp
from jax import lax
from jax.experimental import pallas as pl
from jax.experimental.pallas import tpu as pltpu

N_DEV = 4
M, N = 8192, 4096
K = 2048
CHUNK = M // N_DEV
HALF = N // 2
TILE = 512
N_TILES = CHUNK // TILE
N_HOPS = N_DEV - 1
TILE_G = 1024
GPT = TILE_G // TILE
ATILE = 256


def kernel(x, w_mat):
    def body(x_ref, w_ref, o_ref, z_buf, send_bufs, recv_bufs,
             w_vmem, x_vmem, zt_vmem, vmem_a, vmem_b,
             rs_send_sems, rs_recv_sems, ag_send_sems, ag_recv_sems,
             local_sems, gemm_sems):
        my = lax.axis_index("i")
        left = lax.rem(my - 1 + N_DEV, N_DEV)
        right = lax.rem(my + 1, N_DEV)
        tgt = (right, left)

        def c_rs_send(d, s):
            return lax.rem(my + (s if d else -s) + 2 * N_DEV, N_DEV)

        def c_rs_recv(d, s):
            return lax.rem(my + (s + 1 if d else -s - 1) + 2 * N_DEV, N_DEV)

        def c_ag(d, h):
            return lax.rem(my + (h - 1 if d else 1 - h) + 2 * N_DEV, N_DEV)

        def rows(c, t, size=TILE):
            return pl.ds(c * CHUNK + t * TILE, size)

        def cols(d):
            return pl.ds(d * HALF, HALF)

        rs = [[[None] * N_TILES for _ in range(N_HOPS)] for _ in range(2)]
        ag = [[[None] * N_TILES for _ in range(N_HOPS)] for _ in range(2)]
        for d in (0, 1):
            for s in range(N_HOPS):
                for t in range(N_TILES):
                    if s == 0:
                        src = z_buf.at[rows(c_rs_send(d, 0), t), cols(d)]
                    else:
                        src = send_bufs.at[d, pl.ds(t * TILE, TILE), :]
                    rs[d][s][t] = pltpu.make_async_remote_copy(
                        src_ref=src,
                        dst_ref=recv_bufs.at[d, s, pl.ds(t * TILE, TILE), :],
                        send_sem=rs_send_sems.at[d, s, t],
                        recv_sem=rs_recv_sems.at[d, s, t],
                        device_id=(tgt[d],),
                        device_id_type=pl.DeviceIdType.MESH,
                    )
            for h in range(N_HOPS):
                for t in range(N_TILES):
                    sl = (rows(c_ag(d, h), t), cols(d))
                    ag[d][h][t] = pltpu.make_async_remote_copy(
                        src_ref=o_ref.at[sl],
                        dst_ref=o_ref.at[sl],
                        send_sem=ag_send_sems.at[d, h, t],
                        recv_sem=ag_recv_sems.at[d, h, t],
                        device_id=(tgt[d],),
                        device_id_type=pl.DeviceIdType.MESH,
                    )

        cp_w = pltpu.make_async_copy(w_ref, w_vmem, gemm_sems.at[0])
        cp_w.start()

        barrier = pltpu.get_barrier_semaphore()
        for nbr in (left, right):
            pl.semaphore_signal(barrier, inc=1, device_id=(nbr,),
                                device_id_type=pl.DeviceIdType.MESH)
        pl.semaphore_wait(barrier, 2)
        cp_w.wait()

        def _gemm_tile_at(delta, g, j, send_hop0):
            c = lax.rem(my + delta + 2 * N_DEV, N_DEV)
            off = c * CHUNK + g * TILE_G
            cp_x = pltpu.make_async_copy(
                x_ref.at[pl.ds(off, TILE_G), :], x_vmem, gemm_sems.at[0])
            cp_x.start()
            cp_x.wait()
            zt_vmem[...] = jnp.dot(x_vmem[...], w_vmem[...],
                                   preferred_element_type=jnp.float32)
            cp_z = pltpu.make_async_copy(
                zt_vmem, z_buf.at[pl.ds(off, TILE_G), :], gemm_sems.at[1])
            cp_z.start()
            cp_z.wait()
            if send_hop0:
                @pl.when(j == 0)
                def _():
                    for t in range(GPT):
                        rs[0][0][t].start()
                        rs[1][0][t].start()

                @pl.when(j == 1)
                def _():
                    for t in range(GPT, 2 * GPT):
                        rs[0][0][t].start()
                        rs[1][0][t].start()

        def gemm_own(j, carry):
            off = my * CHUNK + j * TILE
            cp_x = pltpu.make_async_copy(
                x_ref.at[pl.ds(off, TILE), :],
                x_vmem.at[pl.ds(0, TILE), :], gemm_sems.at[0])
            cp_x.start()
            cp_x.wait()
            zt_vmem[pl.ds(0, TILE), :] = jnp.dot(
                x_vmem[pl.ds(0, TILE), :], w_vmem[...],
                preferred_element_type=jnp.float32)
            cp_z = pltpu.make_async_copy(
                zt_vmem.at[pl.ds(0, TILE), :],
                z_buf.at[pl.ds(off, TILE), :], gemm_sems.at[1])
            cp_z.start()
            cp_z.wait()
            for k in range(N_TILES):
                @pl.when(j == k)
                def _(k=k):
                    rs[0][0][k].start()
                    rs[1][0][k].start()
            return carry

        def gemm_body1(j, carry):
            delta = jnp.where(j == 0, -1, 1)
            _gemm_tile_at(delta, 0, j, send_hop0=False)
            return carry

        def gemm_body2(j, carry):
            delta = jnp.where(j == 0, -1, jnp.where(j == 1, 1, 2))
            g = jnp.where(j < 2, 1, j - 2)
            _gemm_tile_at(delta, g, j, send_hop0=False)
            return carry

        lax.fori_loop(0, 4, gemm_own, 0)
        lax.fori_loop(0, 2, gemm_body1, 0)

        def emit_add(s, t):
                for d in (0, 1):
                    rs[d][s][t].wait()
                    c = c_rs_recv(d, s)
                    for p in range(TILE // ATILE):
                        off = pl.ds(t * TILE + p * ATILE, ATILE)
                        zoff = pl.ds(c * CHUNK + t * TILE + p * ATILE, ATILE)
                        cp_a = pltpu.make_async_copy(
                            recv_bufs.at[d, s, off, :],
                            vmem_a, local_sems.at[0])
                        cp_b = pltpu.make_async_copy(
                            z_buf.at[zoff, cols(d)],
                            vmem_b, local_sems.at[1])
                        cp_a.start()
                        cp_b.start()
                        cp_a.wait()
                        cp_b.wait()
                        if s < N_HOPS - 1:
                            vmem_a[...] = vmem_a[...] + vmem_b[...]
                            cp_out = pltpu.make_async_copy(
                                vmem_a, send_bufs.at[d, off, :],
                                local_sems.at[0])
                        else:
                            vmem_a[...] = jnp.maximum(
                                vmem_a[...] + vmem_b[...], 0.0)
                            cp_out = pltpu.make_async_copy(
                                vmem_a, o_ref.at[zoff, cols(d)],
                                local_sems.at[0])
                        cp_out.start()
                        cp_out.wait()
                    if s < N_HOPS - 1:
                        rs[d][s + 1][t].start()
                    else:
                        ag[d][0][t].start()

        emit_add(0, 0)
        emit_add(0, 1)
        lax.fori_loop(0, 4, gemm_body2, 0)
        emit_add(0, 2)
        emit_add(0, 3)
        for s in range(1, N_HOPS):
            for t in range(N_TILES):
                emit_add(s, t)

        for h in range(N_HOPS):
            for t in range(N_TILES):
                for d in (0, 1):
                    ag[d][h][t].wait()
                    if h < N_HOPS - 1:
                        ag[d][h + 1][t].start()

    out, _, _, _ = pl.pallas_call(
        body,
        out_shape=(
            jax.ShapeDtypeStruct((M, N), jnp.float32),
            jax.ShapeDtypeStruct((M, N), jnp.float32),
            jax.ShapeDtypeStruct((2, CHUNK, HALF), jnp.float32),
            jax.ShapeDtypeStruct((2, N_HOPS, CHUNK, HALF), jnp.float32),
        ),
        in_specs=[
            pl.BlockSpec(memory_space=pl.ANY),
            pl.BlockSpec(memory_space=pl.ANY),
        ],
        out_specs=(
            pl.BlockSpec(memory_space=pl.ANY),
            pl.BlockSpec(memory_space=pl.ANY),
            pl.BlockSpec(memory_space=pl.ANY),
            pl.BlockSpec(memory_space=pl.ANY),
        ),
        scratch_shapes=[
            pltpu.VMEM((K, N), jnp.float32),
            pltpu.VMEM((TILE_G, K), jnp.float32),
            pltpu.VMEM((TILE_G, N), jnp.float32),
            pltpu.VMEM((ATILE, HALF), jnp.float32),
            pltpu.VMEM((ATILE, HALF), jnp.float32),
            pltpu.SemaphoreType.DMA((2, N_HOPS, N_TILES)),
            pltpu.SemaphoreType.DMA((2, N_HOPS, N_TILES)),
            pltpu.SemaphoreType.DMA((2, N_HOPS, N_TILES)),
            pltpu.SemaphoreType.DMA((2, N_HOPS, N_TILES)),
            pltpu.SemaphoreType.DMA((2,)),
            pltpu.SemaphoreType.DMA((2,)),
        ],
        compiler_params=pltpu.CompilerParams(
            collective_id=0, vmem_limit_bytes=100 * 1024 * 1024),
    )(x, w_mat)
    return out


# device time: 1195605 ns/iter; 2.1495x vs baseline; 1.0081x over previous
import jax
import jax.numpy as jnp
from jax import lax
from jax.experimental import pallas as pl
from jax.experimental.pallas import tpu as pltpu

N_DEV = 4
M, N = 8192, 4096
K = 2048
CHUNK = M // N_DEV
HALF = N // 2
TILE = 512
N_TILES = CHUNK // TILE
N_HOPS = N_DEV - 1
TILE_G = 1024
GPT = TILE_G // TILE
ATILE = 256


def kernel(x, w_mat):
    def body(x_ref, w_ref, o_ref, z_buf, send_bufs, recv_bufs,
             w_vmem, x_vmem, zt_vmem, vmem_a, vmem_b,
             rs_send_sems, rs_recv_sems, ag_send_sems, ag_recv_sems,
             local_sems, gemm_sems):
        my = lax.axis_index("i")
        left = lax.rem(my - 1 + N_DEV, N_DEV)
        right = lax.rem(my + 1, N_DEV)
        tgt = (right, left)

        def c_rs_send(d, s):
            return lax.rem(my + (s if d else -s) + 2 * N_DEV, N_DEV)

        def c_rs_recv(d, s):
            return lax.rem(my + (s + 1 if d else -s - 1) + 2 * N_DEV, N_DEV)

        def c_ag(d, h):
            return lax.rem(my + (h - 1 if d else 1 - h) + 2 * N_DEV, N_DEV)

        def rows(c, t, size=TILE):
            return pl.ds(c * CHUNK + t * TILE, size)

        def cols(d):
            return pl.ds(d * HALF, HALF)

        rs = [[[None] * N_TILES for _ in range(N_HOPS)] for _ in range(2)]
        ag = [[[None] * N_TILES for _ in range(N_HOPS)] for _ in range(2)]
        for d in (0, 1):
            for s in range(N_HOPS):
                for t in range(N_TILES):
                    if s == 0:
                        src = z_buf.at[rows(c_rs_send(d, 0), t), cols(d)]
                    else:
                        src = send_bufs.at[d, pl.ds(t * TILE, TILE), :]
                    rs[d][s][t] = pltpu.make_async_remote_copy(
                        src_ref=src,
                        dst_ref=recv_bufs.at[d, s, pl.ds(t * TILE, TILE), :],
                        send_sem=rs_send_sems.at[d, s, t],
                        recv_sem=rs_recv_sems.at[d, s, t],
                        device_id=(tgt[d],),
                        device_id_type=pl.DeviceIdType.MESH,
                    )
            for h in range(N_HOPS):
                for t in range(N_TILES):
                    sl = (rows(c_ag(d, h), t), cols(d))
                    ag[d][h][t] = pltpu.make_async_remote_copy(
                        src_ref=o_ref.at[sl],
                        dst_ref=o_ref.at[sl],
                        send_sem=ag_send_sems.at[d, h, t],
                        recv_sem=ag_recv_sems.at[d, h, t],
                        device_id=(tgt[d],),
                        device_id_type=pl.DeviceIdType.MESH,
                    )

        cp_w = pltpu.make_async_copy(w_ref, w_vmem, gemm_sems.at[0])
        cp_w.start()

        barrier = pltpu.get_barrier_semaphore()
        for nbr in (left, right):
            pl.semaphore_signal(barrier, inc=1, device_id=(nbr,),
                                device_id_type=pl.DeviceIdType.MESH)
        pl.semaphore_wait(barrier, 2)
        cp_w.wait()

        def _gemm_tile_at(delta, g, j, send_hop0):
            c = lax.rem(my + delta + 2 * N_DEV, N_DEV)
            off = c * CHUNK + g * TILE_G
            cp_x = pltpu.make_async_copy(
                x_ref.at[pl.ds(off, TILE_G), :], x_vmem, gemm_sems.at[0])
            cp_x.start()
            cp_x.wait()
            zt_vmem[...] = jnp.dot(x_vmem[...], w_vmem[...],
                                   preferred_element_type=jnp.float32)
            cp_z = pltpu.make_async_copy(
                zt_vmem, z_buf.at[pl.ds(off, TILE_G), :], gemm_sems.at[1])
            cp_z.start()
            cp_z.wait()
            if send_hop0:
                @pl.when(j == 0)
                def _():
                    for t in range(GPT):
                        rs[0][0][t].start()
                        rs[1][0][t].start()

                @pl.when(j == 1)
                def _():
                    for t in range(GPT, 2 * GPT):
                        rs[0][0][t].start()
                        rs[1][0][t].start()

        def gemm_own(j, carry):
            off = my * CHUNK + j * TILE
            cp_x = pltpu.make_async_copy(
                x_ref.at[pl.ds(off, TILE), :],
                x_vmem.at[pl.ds(0, TILE), :], gemm_sems.at[0])
            cp_x.start()
            cp_x.wait()
            zt_vmem[pl.ds(0, TILE), :] = jnp.dot(
                x_vmem[pl.ds(0, TILE), :], w_vmem[...],
                preferred_element_type=jnp.float32)
            cp_z = pltpu.make_async_copy(
                zt_vmem.at[pl.ds(0, TILE), :],
                z_buf.at[pl.ds(off, TILE), :], gemm_sems.at[1])
            cp_z.start()
            cp_z.wait()
            for k in range(N_TILES):
                @pl.when(j == k)
                def _(k=k):
                    rs[0][0][k].start()
                    rs[1][0][k].start()
            return carry

        def gemm_body1(j, carry):
            delta = jnp.where(j == 0, -1, 1)
            _gemm_tile_at(delta, 0, j, send_hop0=False)
            return carry

        def gemm_body2(j, carry):
            delta = jnp.where(j == 0, -1, 1)
            _gemm_tile_at(delta, 1, j, send_hop0=False)
            return carry

        def gemm_body3(j, carry):
            _gemm_tile_at(2, j, j, send_hop0=False)
            return carry

        lax.fori_loop(0, 4, gemm_own, 0)
        lax.fori_loop(0, 2, gemm_body1, 0)

        def emit_add(s, t):
                for d in (0, 1):
                    rs[d][s][t].wait()
                    c = c_rs_recv(d, s)
                    for p in range(TILE // ATILE):
                        off = pl.ds(t * TILE + p * ATILE, ATILE)
                        zoff = pl.ds(c * CHUNK + t * TILE + p * ATILE, ATILE)
                        cp_a = pltpu.make_async_copy(
                            recv_bufs.at[d, s, off, :],
                            vmem_a, local_sems.at[0])
                        cp_b = pltpu.make_async_copy(
                            z_buf.at[zoff, cols(d)],
                            vmem_b, local_sems.at[1])
                        cp_a.start()
                        cp_b.start()
                        cp_a.wait()
                        cp_b.wait()
                        if s < N_HOPS - 1:
                            vmem_a[...] = vmem_a[...] + vmem_b[...]
                            cp_out = pltpu.make_async_copy(
                                vmem_a, send_bufs.at[d, off, :],
                                local_sems.at[0])
                        else:
                            vmem_a[...] = jnp.maximum(
                                vmem_a[...] + vmem_b[...], 0.0)
                            cp_out = pltpu.make_async_copy(
                                vmem_a, o_ref.at[zoff, cols(d)],
                                local_sems.at[0])
                        cp_out.start()
                        cp_out.wait()
                    if s < N_HOPS - 1:
                        rs[d][s + 1][t].start()
                    else:
                        ag[d][0][t].start()

        emit_add(0, 0)
        emit_add(0, 1)
        lax.fori_loop(0, 2, gemm_body2, 0)
        emit_add(0, 2)
        emit_add(0, 3)
        lax.fori_loop(0, 2, gemm_body3, 0)
        for s in range(1, N_HOPS):
            for t in range(N_TILES):
                emit_add(s, t)

        for h in range(N_HOPS):
            for t in range(N_TILES):
                for d in (0, 1):
                    ag[d][h][t].wait()
                    if h < N_HOPS - 1:
                        ag[d][h + 1][t].start()

    out, _, _, _ = pl.pallas_call(
        body,
        out_shape=(
            jax.ShapeDtypeStruct((M, N), jnp.float32),
            jax.ShapeDtypeStruct((M, N), jnp.float32),
            jax.ShapeDtypeStruct((2, CHUNK, HALF), jnp.float32),
            jax.ShapeDtypeStruct((2, N_HOPS, CHUNK, HALF), jnp.float32),
        ),
        in_specs=[
            pl.BlockSpec(memory_space=pl.ANY),
            pl.BlockSpec(memory_space=pl.ANY),
        ],
        out_specs=(
            pl.BlockSpec(memory_space=pl.ANY),
            pl.BlockSpec(memory_space=pl.ANY),
            pl.BlockSpec(memory_space=pl.ANY),
            pl.BlockSpec(memory_space=pl.ANY),
        ),
        scratch_shapes=[
            pltpu.VMEM((K, N), jnp.float32),
            pltpu.VMEM((TILE_G, K), jnp.float32),
            pltpu.VMEM((TILE_G, N), jnp.float32),
            pltpu.VMEM((ATILE, HALF), jnp.float32),
            pltpu.VMEM((ATILE, HALF), jnp.float32),
            pltpu.SemaphoreType.DMA((2, N_HOPS, N_TILES)),
            pltpu.SemaphoreType.DMA((2, N_HOPS, N_TILES)),
            pltpu.SemaphoreType.DMA((2, N_HOPS, N_TILES)),
            pltpu.SemaphoreType.DMA((2, N_HOPS, N_TILES)),
            pltpu.SemaphoreType.DMA((2,)),
            pltpu.SemaphoreType.DMA((2,)),
        ],
        compiler_params=pltpu.CompilerParams(
            collective_id=0, vmem_limit_bytes=100 * 1024 * 1024),
    )(x, w_mat)
    return out
